# baseline (speedup 1.0000x reference)
import numpy as np

import jax
import jax.numpy as jnp
from jax import lax
from jax.experimental import pallas as pl
from jax.experimental.pallas import tpu as pltpu

_NOISE_RATIO = 0.2
_LABEL_NOISE_RATIO = 0.2
_NUM_CLASSES = 133
_NQ = 100
_LANES = 128

_T = float(np.float32(_NOISE_RATIO)) * (1 << 23)
assert _T != int(_T)
_KEEP_THRESHOLD_M = int(_T) + 1
_TF = float(np.float32(_LABEL_NOISE_RATIO)) * (1 << 23)
assert _TF != int(_TF)
_FLIP_THRESHOLD_M = int(_TF) + 1

_ROT0 = (13, 15, 26, 6)
_ROT1 = (17, 29, 16, 24)
_PARITY = 0x1BD11BDA


def _rotl(x, r):
    return lax.shift_left(x, jnp.int32(r)) | lax.shift_right_logical(
        x, jnp.int32(32 - r))


def _threefry2x32(k0, k1, x0, x1):
    ks = (k0, k1, k0 ^ k1 ^ jnp.int32(_PARITY))
    x0 = x0 + ks[0]
    x1 = x1 + ks[1]
    for i in range(5):
        rots = _ROT0 if i % 2 == 0 else _ROT1
        for r in rots:
            x0 = x0 + x1
            x1 = _rotl(x1, r)
            x1 = x0 ^ x1
        x0 = x0 + ks[(i + 1) % 3]
        x1 = x1 + ks[(i + 2) % 3] + jnp.int32(i + 1)
    return x0, x1


def _derive(k0, k1, i):
    return _threefry2x32(k0, k1, jnp.int32(0), jnp.int32(i)
                         if isinstance(i, int) else i)


def _bits(k, cnt):
    y0, y1 = _threefry2x32(k[0], k[1], jnp.zeros_like(cnt), cnt)
    return y0 ^ y1


def _mod_small(t, c):
    q = (t.astype(jnp.float32) * jnp.float32(1.0 / c)).astype(jnp.int32)
    r = t - q * jnp.int32(c)
    r = jnp.where(r < 0, r + jnp.int32(c), r)
    r = jnp.where(r >= jnp.int32(c), r - jnp.int32(c), r)
    return r


def _mod32(bits, c):
    hi = lax.shift_right_logical(bits, jnp.int32(16))
    lo = bits & jnp.int32(0xFFFF)
    return _mod_small(hi * jnp.int32((1 << 16) % c) + lo, c)


def _randint(k, cnt, c, lo_add):
    k1 = _derive(k[0], k[1], 0)
    k2 = _derive(k[0], k[1], 1)
    hm = _mod32(_bits(k1, cnt), c)
    lm = _mod32(_bits(k2, cnt), c)
    mult = ((1 << 16) % c) ** 2 % c
    return _mod_small(hm * jnp.int32(mult) + lm, c) + jnp.int32(lo_add)


def kernel(mask_labels, class_labels, num_mp_queries):
    B, num_gt, G = mask_labels.shape
    cdtype = class_labels.dtype
    G_BLK = 32768
    assert G % G_BLK == 0
    assert num_gt <= _NQ

    nmq = jnp.asarray(num_mp_queries, jnp.int32).reshape(1)

    cls_col = jnp.zeros((B, _LANES), jnp.float32)
    cls_col = cls_col.at[:, :num_gt].set(
        class_labels.astype(jnp.float32)).reshape(B, _LANES, 1)

    def body(nmq_ref, x_ref, cls_ref, om_ref, oc_ref, oh_scr):
        b = pl.program_id(0)
        g = pl.program_id(1)

        kb = _derive(jnp.int32(0), jnp.int32(42), b)
        k_noise = _derive(kb[0], kb[1], 1)

        @pl.when(g == 0)
        def _prologue():
            k_idx = _derive(kb[0], kb[1], 0)
            k_flip = _derive(kb[0], kb[1], 2)
            k_newlab = _derive(kb[0], kb[1], 3)

            c = lax.broadcasted_iota(jnp.int32, (_LANES, _LANES), 0)
            extra = _randint(k_idx, c - jnp.int32(num_gt), num_gt, 0)
            off = nmq_ref[0] - jnp.int32(_NQ)
            idx = jnp.where(c < jnp.int32(num_gt), c, extra) + off
            idx = jnp.clip(idx, 0, num_gt - 1)

            lane = lax.broadcasted_iota(jnp.int32, (_LANES, _LANES), 1)
            oh_scr[...] = jnp.where(idx == lane, jnp.float32(1.0),
                                    jnp.float32(0.0))

            sampled = lax.dot_general(
                oh_scr[...], cls_ref[0],
                dimension_numbers=(((1,), (0,)), ((), ())),
                precision=lax.Precision.HIGHEST)
            sampled_i = sampled.astype(jnp.int32)
            flip_m = lax.shift_right_logical(
                _bits(k_flip, c), jnp.int32(9))[:, :1]
            newlab = _randint(k_newlab, c, _NUM_CLASSES,  1)[:, :1]
            oc_ref[0] = jnp.where(flip_m < jnp.int32(_FLIP_THRESHOLD_M),
                                  newlab, sampled_i)

        rows = lax.dot_general(
            oh_scr[...][:_NQ, :num_gt], x_ref[0],
            dimension_numbers=(((1,), (0,)), ((), ())),
            precision=lax.Precision.HIGHEST)

        base = g * jnp.int32(G_BLK)
        q2 = lax.broadcasted_iota(jnp.int32, (_NQ, G_BLK), 0)
        j2 = lax.broadcasted_iota(jnp.int32, (_NQ, G_BLK), 1)
        cnt = q2 * jnp.int32(G) + j2 + base
        m = lax.shift_right_logical(_bits(k_noise, cnt), jnp.int32(9))
        keep = m >= jnp.int32(_KEEP_THRESHOLD_M)

        om_ref[0] = jnp.where(keep, rows, jnp.float32(0.0))

    grid_spec = pltpu.PrefetchScalarGridSpec(
        num_scalar_prefetch=1,
        grid=(B, G // G_BLK),
        in_specs=[
            pl.BlockSpec((1, num_gt, G_BLK), lambda b, g, *_: (b, 0, g)),
            pl.BlockSpec((1, _LANES, 1), lambda b, g, *_: (b, 0, 0)),
        ],
        out_specs=[
            pl.BlockSpec((1, _NQ, G_BLK), lambda b, g, *_: (b, 0, g)),
            pl.BlockSpec((1, _LANES, 1), lambda b, g, *_: (b, 0, 0)),
        ],
        scratch_shapes=[pltpu.VMEM((_LANES, _LANES), jnp.float32)],
    )

    out_masks, out_cls = pl.pallas_call(
        body,
        grid_spec=grid_spec,
        compiler_params=pltpu.CompilerParams(
            dimension_semantics=("parallel", "arbitrary")),
        out_shape=[
            jax.ShapeDtypeStruct((B, _NQ, G), mask_labels.dtype),
            jax.ShapeDtypeStruct((B, _LANES, 1), jnp.int32),
        ],
    )(nmq, mask_labels, cls_col)

    return (out_masks, out_cls[:, :_NQ, 0].astype(cdtype))


# speedup vs baseline: 1.1710x; 1.1710x over previous
import numpy as np

import jax
import jax.numpy as jnp
from jax import lax
from jax.experimental import pallas as pl
from jax.experimental.pallas import tpu as pltpu

_NOISE_RATIO = 0.2
_LABEL_NOISE_RATIO = 0.2
_NUM_CLASSES = 133
_NQ = 100
_LANES = 128

_T = float(np.float32(_NOISE_RATIO)) * (1 << 23)
assert _T != int(_T)
_KEEP_THRESHOLD_M = int(_T) + 1
_TF = float(np.float32(_LABEL_NOISE_RATIO)) * (1 << 23)
assert _TF != int(_TF)
_FLIP_THRESHOLD_M = int(_TF) + 1

_ROT0 = (13, 15, 26, 6)
_ROT1 = (17, 29, 16, 24)
_PARITY = 0x1BD11BDA


def _rotl(x, r):
    return lax.shift_left(x, jnp.int32(r)) | lax.shift_right_logical(
        x, jnp.int32(32 - r))


def _threefry2x32(k0, k1, x0, x1):
    ks = (k0, k1, k0 ^ k1 ^ jnp.int32(_PARITY))
    x0 = x0 + ks[0]
    x1 = x1 + ks[1]
    for i in range(5):
        rots = _ROT0 if i % 2 == 0 else _ROT1
        for r in rots:
            x0 = x0 + x1
            x1 = _rotl(x1, r)
            x1 = x0 ^ x1
        x0 = x0 + ks[(i + 1) % 3]
        x1 = x1 + ks[(i + 2) % 3] + jnp.int32(i + 1)
    return x0, x1


def _derive(k0, k1, i):
    return _threefry2x32(k0, k1, jnp.int32(0), jnp.int32(i)
                         if isinstance(i, int) else i)


def _bits(k, cnt):
    y0, y1 = _threefry2x32(k[0], k[1], jnp.zeros_like(cnt), cnt)
    return y0 ^ y1


def _mod_small(t, c):
    q = (t.astype(jnp.float32) * jnp.float32(1.0 / c)).astype(jnp.int32)
    r = t - q * jnp.int32(c)
    r = jnp.where(r < 0, r + jnp.int32(c), r)
    r = jnp.where(r >= jnp.int32(c), r - jnp.int32(c), r)
    return r


def _mod32(bits, c):
    hi = lax.shift_right_logical(bits, jnp.int32(16))
    lo = bits & jnp.int32(0xFFFF)
    return _mod_small(hi * jnp.int32((1 << 16) % c) + lo, c)


def _randint(k, cnt, c, lo_add):
    k1 = _derive(k[0], k[1], 0)
    k2 = _derive(k[0], k[1], 1)
    hm = _mod32(_bits(k1, cnt), c)
    lm = _mod32(_bits(k2, cnt), c)
    mult = ((1 << 16) % c) ** 2 % c
    return _mod_small(hm * jnp.int32(mult) + lm, c) + jnp.int32(lo_add)


def kernel(mask_labels, class_labels, num_mp_queries):
    B, num_gt, G = mask_labels.shape
    cdtype = class_labels.dtype
    G_BLK = 16384
    assert G % G_BLK == 0
    assert num_gt <= _NQ

    nmq = jnp.asarray(num_mp_queries, jnp.int32).reshape(1)

    cls_col = jnp.zeros((B, _LANES), jnp.float32)
    cls_col = cls_col.at[:, :num_gt].set(
        class_labels.astype(jnp.float32)).reshape(B, _LANES, 1)

    def body(nmq_ref, x_ref, cls_ref, om_ref, oc_ref, oh_scr):
        b = pl.program_id(0)
        g = pl.program_id(1)

        kb = _derive(jnp.int32(0), jnp.int32(42), b)
        k_noise = _derive(kb[0], kb[1], 1)

        @pl.when(g == 0)
        def _prologue():
            k_idx = _derive(kb[0], kb[1], 0)
            k_flip = _derive(kb[0], kb[1], 2)
            k_newlab = _derive(kb[0], kb[1], 3)

            c = lax.broadcasted_iota(jnp.int32, (_LANES, _LANES), 0)
            extra = _randint(k_idx, c - jnp.int32(num_gt), num_gt, 0)
            off = nmq_ref[0] - jnp.int32(_NQ)
            idx = jnp.where(c < jnp.int32(num_gt), c, extra) + off
            idx = jnp.clip(idx, 0, num_gt - 1)

            lane = lax.broadcasted_iota(jnp.int32, (_LANES, _LANES), 1)
            oh_scr[...] = jnp.where(idx == lane, jnp.float32(1.0),
                                    jnp.float32(0.0))

            sampled = lax.dot_general(
                oh_scr[...], cls_ref[0],
                dimension_numbers=(((1,), (0,)), ((), ())),
                precision=lax.Precision.HIGHEST)
            sampled_i = sampled.astype(jnp.int32)
            flip_m = lax.shift_right_logical(
                _bits(k_flip, c), jnp.int32(9))[:, :1]
            newlab = _randint(k_newlab, c, _NUM_CLASSES,  1)[:, :1]
            oc_ref[0] = jnp.where(flip_m < jnp.int32(_FLIP_THRESHOLD_M),
                                  newlab, sampled_i)

        rows = lax.dot_general(
            oh_scr[...][:_NQ, :num_gt], x_ref[0],
            dimension_numbers=(((1,), (0,)), ((), ())),
            precision=lax.Precision.HIGHEST)

        base = g * jnp.int32(G_BLK)
        q2 = lax.broadcasted_iota(jnp.int32, (_NQ, G_BLK), 0)
        j2 = lax.broadcasted_iota(jnp.int32, (_NQ, G_BLK), 1)
        cnt = q2 * jnp.int32(G) + j2 + base
        m = lax.shift_right_logical(_bits(k_noise, cnt), jnp.int32(9))
        keep = m >= jnp.int32(_KEEP_THRESHOLD_M)

        om_ref[0] = jnp.where(keep, rows, jnp.float32(0.0))

    grid_spec = pltpu.PrefetchScalarGridSpec(
        num_scalar_prefetch=1,
        grid=(B, G // G_BLK),
        in_specs=[
            pl.BlockSpec((1, num_gt, G_BLK), lambda b, g, *_: (b, 0, g)),
            pl.BlockSpec((1, _LANES, 1), lambda b, g, *_: (b, 0, 0)),
        ],
        out_specs=[
            pl.BlockSpec((1, _NQ, G_BLK), lambda b, g, *_: (b, 0, g)),
            pl.BlockSpec((1, _LANES, 1), lambda b, g, *_: (b, 0, 0)),
        ],
        scratch_shapes=[pltpu.VMEM((_LANES, _LANES), jnp.float32)],
    )

    out_masks, out_cls = pl.pallas_call(
        body,
        grid_spec=grid_spec,
        compiler_params=pltpu.CompilerParams(
            dimension_semantics=("parallel", "arbitrary")),
        out_shape=[
            jax.ShapeDtypeStruct((B, _NQ, G), mask_labels.dtype),
            jax.ShapeDtypeStruct((B, _LANES, 1), jnp.int32),
        ],
    )(nmq, mask_labels, cls_col)

    return (out_masks, out_cls[:, :_NQ, 0].astype(cdtype))
